# baseline (speedup 1.0000x reference)
import functools

import jax
import jax.numpy as jnp
from jax import lax
from jax.experimental import pallas as pl
from jax.experimental.pallas import tpu as pltpu
from jax.experimental.pallas import tpu_sc as plsc

VOCAB = 1000000
EMBED_DIM = 64
BATCH = 16384
HIST = 200

NUM_CORES = 2
NUM_SUBCORES = 16
NUM_WORKERS = NUM_CORES * NUM_SUBCORES

B = BATCH * HIST
B_PER_W = B // NUM_WORKERS
CHUNK = 1024
N_CHUNKS = B_PER_W // CHUNK

assert B_PER_W * NUM_WORKERS == B
assert N_CHUNKS * CHUNK == B_PER_W


def _make_kernel():
    mesh = plsc.VectorSubcoreMesh(core_axis_name="c", subcore_axis_name="s")

    @functools.partial(
        pl.kernel,
        mesh=mesh,
        out_type=jax.ShapeDtypeStruct((B, EMBED_DIM), jnp.float32),
        scratch_types=[
            pltpu.VMEM((CHUNK,), jnp.int32),
            pltpu.VMEM((CHUNK, EMBED_DIM), jnp.float32),
            pltpu.SemaphoreType.DMA,
        ],
        compiler_params=pltpu.CompilerParams(use_tc_tiling_on_sc=False),
    )
    def gather_kernel(idx_hbm, table_hbm, out_hbm, idx_v, rows_v, sem):
        wid = lax.axis_index("s") * NUM_CORES + lax.axis_index("c")
        base = wid * B_PER_W

        def body(g, carry):
            off = base + g * CHUNK
            pltpu.sync_copy(idx_hbm.at[pl.ds(off, CHUNK)], idx_v)
            pltpu.async_copy(table_hbm.at[idx_v], rows_v, sem).wait()
            pltpu.sync_copy(rows_v, out_hbm.at[pl.ds(off, CHUNK)])
            return carry

        lax.fori_loop(0, N_CHUNKS, body, 0)

    return gather_kernel


_gather = _make_kernel()


def kernel(product_ids, table):
    idx = product_ids.reshape(B).astype(jnp.int32)
    rows = _gather(idx, table)
    return rows.reshape(BATCH, HIST, EMBED_DIM)


# speedup vs baseline: 1.0311x; 1.0311x over previous
import functools

import jax
import jax.numpy as jnp
from jax import lax
from jax.experimental import pallas as pl
from jax.experimental.pallas import tpu as pltpu
from jax.experimental.pallas import tpu_sc as plsc

VOCAB = 1000000
EMBED_DIM = 64
BATCH = 16384
HIST = 200

NUM_CORES = 2
NUM_SUBCORES = 16
NUM_WORKERS = NUM_CORES * NUM_SUBCORES

B = BATCH * HIST
B_PER_W = B // NUM_WORKERS
CHUNK = 800
N_CHUNKS = B_PER_W // CHUNK

assert B_PER_W * NUM_WORKERS == B
assert N_CHUNKS * CHUNK == B_PER_W
assert N_CHUNKS % 4 == 0 and N_CHUNKS >= 8
assert CHUNK % 8 == 0
assert 2 * CHUNK * EMBED_DIM * 4 + 4 * CHUNK * 4 < 524284


def _make_kernel():
    mesh = plsc.VectorSubcoreMesh(core_axis_name="c", subcore_axis_name="s")

    @functools.partial(
        pl.kernel,
        mesh=mesh,
        out_type=jax.ShapeDtypeStruct((B, EMBED_DIM), jnp.float32),
        scratch_types=[
            pltpu.VMEM((CHUNK,), jnp.int32),
            pltpu.VMEM((CHUNK,), jnp.int32),
            pltpu.VMEM((CHUNK,), jnp.int32),
            pltpu.VMEM((CHUNK,), jnp.int32),
            pltpu.VMEM((CHUNK, EMBED_DIM), jnp.float32),
            pltpu.VMEM((CHUNK, EMBED_DIM), jnp.float32),
            pltpu.SemaphoreType.DMA,
            pltpu.SemaphoreType.DMA,
            pltpu.SemaphoreType.DMA,
            pltpu.SemaphoreType.DMA,
            pltpu.SemaphoreType.DMA,
            pltpu.SemaphoreType.DMA,
            pltpu.SemaphoreType.DMA,
            pltpu.SemaphoreType.DMA,
        ],
        compiler_params=pltpu.CompilerParams(use_tc_tiling_on_sc=False),
    )
    def gather_kernel(idx_hbm, table_hbm, out_hbm,
                      i0, i1, i2, i3, r0, r1,
                      si0, si1, si2, si3, sg0, sg1, sw0, sw1):
        ibuf = [i0, i1, i2, i3]
        rows = [r0, r1]
        sem_i = [si0, si1, si2, si3]
        sem_g = [sg0, sg1]
        sem_w = [sw0, sw1]

        wid = lax.axis_index("s") * NUM_CORES + lax.axis_index("c")
        base = wid * B_PER_W

        def idx_start(g, b):
            pltpu.async_copy(idx_hbm.at[pl.ds(base + g * CHUNK, CHUNK)],
                             ibuf[b], sem_i[b])

        def idx_wait(b):
            pltpu.make_async_copy(idx_hbm.at[pl.ds(base, CHUNK)],
                                  ibuf[b], sem_i[b]).wait()

        def gather_start(bi, br):
            pltpu.async_copy(table_hbm.at[ibuf[bi]], rows[br], sem_g[br])

        def gather_wait(bi, br):
            pltpu.make_async_copy(table_hbm.at[ibuf[bi]],
                                  rows[br], sem_g[br]).wait()

        def wb_start(g, b):
            pltpu.async_copy(rows[b],
                             out_hbm.at[pl.ds(base + g * CHUNK, CHUNK)],
                             sem_w[b])

        def wb_wait(b):
            pltpu.make_async_copy(rows[b],
                                  out_hbm.at[pl.ds(base, CHUNK)],
                                  sem_w[b]).wait()

        for j in range(4):
            idx_start(j, j)
        idx_wait(0)
        gather_start(0, 0)
        for j in range(4):
            gather_wait(j % 4, j % 2)
            wb_start(j, j % 2)
            if j > 0:
                wb_wait((j - 1) % 2)
            idx_wait((j + 1) % 4)
            gather_start((j + 1) % 4, (j + 1) % 2)
            idx_start(j + 4, j)

        def quad(k, carry):
            for j in range(4):
                g = 4 * k + j
                gather_wait(j % 4, j % 2)
                wb_start(g, j % 2)
                wb_wait((j + 1) % 2)
                idx_wait((j + 1) % 4)
                gather_start((j + 1) % 4, (j + 1) % 2)
                idx_start(jnp.minimum(g + 4, N_CHUNKS - 1), j)
            return carry

        lax.fori_loop(1, N_CHUNKS // 4, quad, 0)

        wb_wait((N_CHUNKS - 1) % 2)
        gather_wait(0, 0)
        idx_wait(1)
        idx_wait(2)
        idx_wait(3)

    return gather_kernel


_gather = _make_kernel()


def kernel(product_ids, table):
    idx = product_ids.reshape(B).astype(jnp.int32)
    rows = _gather(idx, table)
    return rows.reshape(BATCH, HIST, EMBED_DIM)


# speedup vs baseline: 1.3303x; 1.2902x over previous
import functools

import jax
import jax.numpy as jnp
from jax import lax
from jax.experimental import pallas as pl
from jax.experimental.pallas import tpu as pltpu
from jax.experimental.pallas import tpu_sc as plsc

VOCAB = 1000000
EMBED_DIM = 64
BATCH = 16384
HIST = 200
PAD_DIM = 128

NUM_CORES = 2
NUM_SUBCORES = 16
NUM_WORKERS = NUM_CORES * NUM_SUBCORES

B = BATCH * HIST
B_PER_W = B // NUM_WORKERS
CHUNK = 400
N_CHUNKS = B_PER_W // CHUNK

assert B_PER_W * NUM_WORKERS == B
assert N_CHUNKS * CHUNK == B_PER_W
assert N_CHUNKS % 4 == 0 and N_CHUNKS >= 8
assert CHUNK % 8 == 0
assert 2 * CHUNK * PAD_DIM * 4 + 4 * CHUNK * 4 < 524284


def _make_kernel():
    mesh = plsc.VectorSubcoreMesh(core_axis_name="c", subcore_axis_name="s")

    @functools.partial(
        pl.kernel,
        mesh=mesh,
        out_type=jax.ShapeDtypeStruct((B, PAD_DIM), jnp.float32),
        scratch_types=[
            pltpu.VMEM((CHUNK,), jnp.int32),
            pltpu.VMEM((CHUNK,), jnp.int32),
            pltpu.VMEM((CHUNK,), jnp.int32),
            pltpu.VMEM((CHUNK,), jnp.int32),
            pltpu.VMEM((CHUNK, PAD_DIM), jnp.float32),
            pltpu.VMEM((CHUNK, PAD_DIM), jnp.float32),
            pltpu.SemaphoreType.DMA,
            pltpu.SemaphoreType.DMA,
            pltpu.SemaphoreType.DMA,
            pltpu.SemaphoreType.DMA,
            pltpu.SemaphoreType.DMA,
            pltpu.SemaphoreType.DMA,
            pltpu.SemaphoreType.DMA,
            pltpu.SemaphoreType.DMA,
        ],
        compiler_params=pltpu.CompilerParams(use_tc_tiling_on_sc=True),
    )
    def gather_kernel(idx_hbm, table_hbm, out_hbm,
                      i0, i1, i2, i3, r0, r1,
                      si0, si1, si2, si3, sg0, sg1, sw0, sw1):
        ibuf = [i0, i1, i2, i3]
        rows = [r0, r1]
        sem_i = [si0, si1, si2, si3]
        sem_g = [sg0, sg1]
        sem_w = [sw0, sw1]

        wid = lax.axis_index("s") * NUM_CORES + lax.axis_index("c")
        base = wid * B_PER_W

        def idx_start(g, b):
            pltpu.async_copy(idx_hbm.at[pl.ds(base + g * CHUNK, CHUNK)],
                             ibuf[b], sem_i[b])

        def idx_wait(b):
            pltpu.make_async_copy(idx_hbm.at[pl.ds(base, CHUNK)],
                                  ibuf[b], sem_i[b]).wait()

        def gather_start(bi, br):
            pltpu.async_copy(table_hbm.at[ibuf[bi]], rows[br], sem_g[br])

        def gather_wait(bi, br):
            pltpu.make_async_copy(table_hbm.at[ibuf[bi]],
                                  rows[br], sem_g[br]).wait()

        def wb_start(g, b):
            pltpu.async_copy(rows[b],
                             out_hbm.at[pl.ds(base + g * CHUNK, CHUNK)],
                             sem_w[b])

        def wb_wait(b):
            pltpu.make_async_copy(rows[b],
                                  out_hbm.at[pl.ds(base, CHUNK)],
                                  sem_w[b]).wait()

        for j in range(4):
            idx_start(j, j)
        idx_wait(0)
        gather_start(0, 0)
        for j in range(4):
            gather_wait(j % 4, j % 2)
            wb_start(j, j % 2)
            if j > 0:
                wb_wait((j - 1) % 2)
            idx_wait((j + 1) % 4)
            gather_start((j + 1) % 4, (j + 1) % 2)
            idx_start(j + 4, j)

        def quad(k, carry):
            for j in range(4):
                g = 4 * k + j
                gather_wait(j % 4, j % 2)
                wb_start(g, j % 2)
                wb_wait((j + 1) % 2)
                idx_wait((j + 1) % 4)
                gather_start((j + 1) % 4, (j + 1) % 2)
                idx_start(jnp.minimum(g + 4, N_CHUNKS - 1), j)
            return carry

        lax.fori_loop(1, N_CHUNKS // 4, quad, 0)

        wb_wait((N_CHUNKS - 1) % 2)
        gather_wait(0, 0)
        idx_wait(1)
        idx_wait(2)
        idx_wait(3)

    return gather_kernel


_gather = _make_kernel()


def kernel(product_ids, table):
    idx = product_ids.reshape(B).astype(jnp.int32)
    table_padded = jnp.concatenate(
        [table, jnp.zeros((VOCAB, PAD_DIM - EMBED_DIM), jnp.float32)], axis=1)
    rows = _gather(idx, table_padded)
    return rows[:, :EMBED_DIM].reshape(BATCH, HIST, EMBED_DIM)
